# baseline (speedup 1.0000x reference)
import functools

import jax
import jax.numpy as jnp
from jax.experimental import pallas as pl
from jax.experimental.pallas import tpu as pltpu


def _heads(adj_blk, h_rows, h_full, aw, ab):
    hid = h_rows.shape[1]
    heads = aw.shape[0]
    src_all = jax.lax.dot_general(
        h_rows, aw[:, :hid], (((1,), (1,)), ((), ())),
        preferred_element_type=jnp.float32)
    dst_all = jax.lax.dot_general(
        aw[:, hid:], h_full, (((1,), (1,)), ((), ())),
        preferred_element_type=jnp.float32)
    outs = []
    for h in range(heads):
        dstb = dst_all[h:h + 1, :] + ab[0, h]
        src = src_all[:, h:h + 1]
        tmax = src + jnp.max(dstb)
        s = jnp.maximum(tmax, 0.01 * tmax)
        t = src + dstb
        e = jnp.maximum(t, 0.01 * t)
        ex = jnp.exp(e - s) * adj_blk
        denom = jnp.sum(ex, axis=1, keepdims=True)
        safe = jnp.where(denom > 0, denom, 1.0)
        alpha = ex / safe
        agg = jnp.dot(alpha, h_full, preferred_element_type=jnp.float32)
        outs.append(h_rows + agg)
    return outs


def _h1_kernel(adj_ref, feats_ref, w1_ref, b1_ref, out_ref):
    af = jnp.dot(adj_ref[...], feats_ref[...],
                 preferred_element_type=jnp.float32)
    out_ref[...] = (jnp.dot(af, w1_ref[...],
                            preferred_element_type=jnp.float32) + b1_ref[...])


def _l1_kernel(adj_ref, h1_ref, aw_ref, ab_ref, w2_ref, b2_ref, out_ref):
    i = pl.program_id(0)
    blk = adj_ref.shape[0]
    h_full = h1_ref[...]
    h_rows = h1_ref[pl.ds(i * blk, blk), :]
    outs = _heads(adj_ref[...], h_rows, h_full, aw_ref[...], ab_ref[...])
    cat = jnp.concatenate(outs, axis=1)
    act = jnp.where(cat > 0, cat, jnp.exp(cat) - 1.0)
    out_ref[...] = (jnp.dot(act, w2_ref[...],
                            preferred_element_type=jnp.float32) + b2_ref[...])


def _l2_kernel(adj_ref, h2_ref, aw_ref, ab_ref, ow_ref, ob_ref, out_ref,
               acc_ref, *, n_total):
    i = pl.program_id(0)
    blk = adj_ref.shape[0]
    h_full = h2_ref[...]
    h_rows = h2_ref[pl.ds(i * blk, blk), :]
    outs = _heads(adj_ref[...], h_rows, h_full, aw_ref[...], ab_ref[...])
    avg = outs[0]
    for o in outs[1:]:
        avg = avg + o
    avg = avg * (1.0 / len(outs))
    act = jnp.where(avg > 0, avg, jnp.exp(avg) - 1.0)
    part = jnp.sum(act, axis=0, keepdims=True)

    @pl.when(i == 0)
    def _():
        acc_ref[...] = jnp.zeros_like(acc_ref)

    acc_ref[...] += part

    @pl.when(i == pl.num_programs(0) - 1)
    def _():
        avgd = acc_ref[...] * (1.0 / n_total)
        out_ref[...] = (jnp.dot(avgd, ow_ref[...],
                                preferred_element_type=jnp.float32)
                        + ob_ref[...])


def kernel(adjacency_matrix, feats, W1_w, W1_b, a1_w, a1_b, W2_w, W2_b,
           a2_w, a2_b, out_w, out_b):
    n = adjacency_matrix.shape[0]
    d_feat = feats.shape[1]
    hid = W1_w.shape[1]
    heads = a1_w.shape[0]
    blk = min(256, n)
    grid = (n // blk,)

    b1 = W1_b.reshape(1, hid)
    b2 = W2_b.reshape(1, hid)
    ab1 = a1_b.reshape(1, heads)
    ab2 = a2_b.reshape(1, heads)
    ob = out_b.reshape(1, 1)

    full = lambda shape: pl.BlockSpec(shape, lambda i: (0,) * len(shape))
    rows = lambda cols: pl.BlockSpec((blk, cols), lambda i: (i, 0))

    h1 = pl.pallas_call(
        _h1_kernel,
        grid=grid,
        in_specs=[rows(n), full((n, d_feat)), full((d_feat, hid)),
                  full((1, hid))],
        out_specs=rows(hid),
        out_shape=jax.ShapeDtypeStruct((n, hid), jnp.float32),
        compiler_params=pltpu.CompilerParams(
            dimension_semantics=("parallel",)),
    )(adjacency_matrix, feats, W1_w, b1)

    h2 = pl.pallas_call(
        _l1_kernel,
        grid=grid,
        in_specs=[rows(n), full((n, hid)), full((heads, 2 * hid)),
                  full((1, heads)), full((heads * hid, hid)), full((1, hid))],
        out_specs=rows(hid),
        out_shape=jax.ShapeDtypeStruct((n, hid), jnp.float32),
        compiler_params=pltpu.CompilerParams(
            dimension_semantics=("parallel",)),
    )(adjacency_matrix, h1, a1_w, ab1, W2_w, b2)

    out = pl.pallas_call(
        functools.partial(_l2_kernel, n_total=n),
        grid=grid,
        in_specs=[rows(n), full((n, hid)), full((heads, 2 * hid)),
                  full((1, heads)), full((hid, 1)), full((1, 1))],
        out_specs=full((1, 1)),
        out_shape=jax.ShapeDtypeStruct((1, 1), jnp.float32),
        scratch_shapes=[pltpu.VMEM((1, hid), jnp.float32)],
    )(adjacency_matrix, h2, a2_w, ab2, out_w, ob)

    return out.reshape(1)


# speedup vs baseline: 1.1365x; 1.1365x over previous
import functools

import jax
import jax.numpy as jnp
from jax.experimental import pallas as pl
from jax.experimental.pallas import tpu as pltpu


def _heads(adj_blk, h_rows, h_full, aw, ab):
    hid = h_rows.shape[1]
    heads = aw.shape[0]
    src_all = jax.lax.dot_general(
        h_rows, aw[:, :hid], (((1,), (1,)), ((), ())),
        preferred_element_type=jnp.float32)
    dst_all = jax.lax.dot_general(
        aw[:, hid:], h_full, (((1,), (1,)), ((), ())),
        preferred_element_type=jnp.float32)
    log2e = 1.4426950408889634
    outs = []
    for h in range(heads):
        dstb = (dst_all[h:h + 1, :] + ab[0, h]) * log2e
        src = src_all[:, h:h + 1] * log2e
        tmax = src + jnp.max(dstb)
        s = jnp.maximum(tmax, 0.01 * tmax)
        t = src + dstb
        e = jnp.maximum(t, 0.01 * t)
        ex = jnp.exp2(e - s) * adj_blk
        denom = jnp.sum(ex, axis=1, keepdims=True)
        safe = jnp.where(denom > 0, denom, 1.0)
        alpha = ex / safe
        agg = jnp.dot(alpha, h_full, preferred_element_type=jnp.float32)
        outs.append(h_rows + agg)
    return outs


def _h1_kernel(adj_ref, feats_ref, w1_ref, b1_ref, out_ref):
    af = jnp.dot(adj_ref[...], feats_ref[...],
                 preferred_element_type=jnp.float32)
    out_ref[...] = (jnp.dot(af, w1_ref[...],
                            preferred_element_type=jnp.float32) + b1_ref[...])


def _l1_kernel(adj_ref, h1_ref, aw_ref, ab_ref, w2_ref, b2_ref, out_ref):
    i = pl.program_id(0)
    blk = adj_ref.shape[0]
    h_full = h1_ref[...]
    h_rows = h1_ref[pl.ds(i * blk, blk), :]
    outs = _heads(adj_ref[...], h_rows, h_full, aw_ref[...], ab_ref[...])
    cat = jnp.concatenate(outs, axis=1)
    act = jnp.where(cat > 0, cat, jnp.exp(cat) - 1.0)
    out_ref[...] = (jnp.dot(act, w2_ref[...],
                            preferred_element_type=jnp.float32) + b2_ref[...])


def _l2_kernel(adj_ref, h2_ref, aw_ref, ab_ref, ow_ref, ob_ref, out_ref,
               acc_ref, *, n_total):
    i = pl.program_id(0)
    blk = adj_ref.shape[0]
    h_full = h2_ref[...]
    h_rows = h2_ref[pl.ds(i * blk, blk), :]
    outs = _heads(adj_ref[...], h_rows, h_full, aw_ref[...], ab_ref[...])
    avg = outs[0]
    for o in outs[1:]:
        avg = avg + o
    avg = avg * (1.0 / len(outs))
    act = jnp.where(avg > 0, avg, jnp.exp(avg) - 1.0)
    part = jnp.sum(act, axis=0, keepdims=True)

    @pl.when(i == 0)
    def _():
        acc_ref[...] = jnp.zeros_like(acc_ref)

    acc_ref[...] += part

    @pl.when(i == pl.num_programs(0) - 1)
    def _():
        avgd = acc_ref[...] * (1.0 / n_total)
        out_ref[...] = (jnp.dot(avgd, ow_ref[...],
                                preferred_element_type=jnp.float32)
                        + ob_ref[...])


def kernel(adjacency_matrix, feats, W1_w, W1_b, a1_w, a1_b, W2_w, W2_b,
           a2_w, a2_b, out_w, out_b):
    n = adjacency_matrix.shape[0]
    d_feat = feats.shape[1]
    hid = W1_w.shape[1]
    heads = a1_w.shape[0]
    blk = min(512, n)
    grid = (n // blk,)

    b1 = W1_b.reshape(1, hid)
    b2 = W2_b.reshape(1, hid)
    ab1 = a1_b.reshape(1, heads)
    ab2 = a2_b.reshape(1, heads)
    ob = out_b.reshape(1, 1)

    full = lambda shape: pl.BlockSpec(shape, lambda i: (0,) * len(shape))
    rows = lambda cols: pl.BlockSpec((blk, cols), lambda i: (i, 0))

    h1 = pl.pallas_call(
        _h1_kernel,
        grid=grid,
        in_specs=[rows(n), full((n, d_feat)), full((d_feat, hid)),
                  full((1, hid))],
        out_specs=rows(hid),
        out_shape=jax.ShapeDtypeStruct((n, hid), jnp.float32),
        compiler_params=pltpu.CompilerParams(
            dimension_semantics=("parallel",)),
    )(adjacency_matrix, feats, W1_w, b1)

    h2 = pl.pallas_call(
        _l1_kernel,
        grid=grid,
        in_specs=[rows(n), full((n, hid)), full((heads, 2 * hid)),
                  full((1, heads)), full((heads * hid, hid)), full((1, hid))],
        out_specs=rows(hid),
        out_shape=jax.ShapeDtypeStruct((n, hid), jnp.float32),
        compiler_params=pltpu.CompilerParams(
            dimension_semantics=("parallel",)),
    )(adjacency_matrix, h1, a1_w, ab1, W2_w, b2)

    out = pl.pallas_call(
        functools.partial(_l2_kernel, n_total=n),
        grid=grid,
        in_specs=[rows(n), full((n, hid)), full((heads, 2 * hid)),
                  full((1, heads)), full((hid, 1)), full((1, 1))],
        out_specs=full((1, 1)),
        out_shape=jax.ShapeDtypeStruct((1, 1), jnp.float32),
        scratch_shapes=[pltpu.VMEM((1, hid), jnp.float32)],
    )(adjacency_matrix, h2, a2_w, ab2, out_w, ob)

    return out.reshape(1)
